# Initial kernel scaffold; baseline (speedup 1.0000x reference)
import jax
import jax.numpy as jnp
from jax.experimental import pallas as pl


def kernel(x, table):
    raise NotImplementedError("write your pallas kernel here")


# speedup vs baseline: 4.1030x; 4.1030x over previous
import functools

import jax
import jax.numpy as jnp
from jax import lax
from jax.experimental import pallas as pl
from jax.experimental.pallas import tpu as pltpu
from jax.experimental.pallas import tpu_sc as plsc

VOCAB = 100000
EMBED_DIM = 64
BATCH = 4096
HIST = 50

NUM_CORES = 2
NUM_SUBCORES = 16
NW = NUM_CORES * NUM_SUBCORES
TOTAL = BATCH * HIST
BPW = TOTAL // NW
CHUNK = 128
CHUNKS = BPW // CHUNK


def _make_gather():
    mesh = plsc.VectorSubcoreMesh(core_axis_name="c", subcore_axis_name="s")

    @functools.partial(
        pl.kernel,
        mesh=mesh,
        out_type=jax.ShapeDtypeStruct((TOTAL, EMBED_DIM), jnp.float32),
        scratch_types=[
            pltpu.VMEM((CHUNKS, CHUNK), jnp.int32),
            pltpu.VMEM((CHUNK, EMBED_DIM), jnp.float32),
            pltpu.SemaphoreType.DMA,
        ],
        compiler_params=pltpu.CompilerParams(use_tc_tiling_on_sc=False),
    )
    def gather_kernel(idx_hbm, table_hbm, out_hbm, idx_v, rows_v, sem):
        wid = lax.axis_index("s") * NUM_CORES + lax.axis_index("c")
        base = wid * BPW
        pltpu.sync_copy(idx_hbm.at[wid], idx_v)

        def body(j, carry):
            pltpu.async_copy(table_hbm.at[idx_v.at[j]], rows_v, sem).wait()
            pltpu.sync_copy(rows_v, out_hbm.at[pl.ds(base + j * CHUNK, CHUNK)])
            return carry

        lax.fori_loop(0, CHUNKS, body, 0)

    return gather_kernel


_gather = _make_gather()


def kernel(x, table):
    idx = x.reshape(NW, CHUNKS, CHUNK).astype(jnp.int32)
    out = _gather(idx, table)
    return out.reshape(BATCH, HIST, EMBED_DIM)


# speedup vs baseline: 4.6746x; 1.1393x over previous
import functools

import jax
import jax.numpy as jnp
from jax import lax
from jax.experimental import pallas as pl
from jax.experimental.pallas import tpu as pltpu
from jax.experimental.pallas import tpu_sc as plsc

VOCAB = 100000
EMBED_DIM = 64
BATCH = 4096
HIST = 50

NUM_CORES = 2
NUM_SUBCORES = 16
NW = NUM_CORES * NUM_SUBCORES
TOTAL = BATCH * HIST
BPW = TOTAL // NW
CHUNK = 128
CHUNKS = BPW // CHUNK
K = 5
MROWS = K * CHUNK
NMACRO = CHUNKS // K


def _make_gather():
    mesh = plsc.VectorSubcoreMesh(core_axis_name="c", subcore_axis_name="s")

    @functools.partial(
        pl.kernel,
        mesh=mesh,
        out_type=jax.ShapeDtypeStruct((TOTAL, EMBED_DIM), jnp.float32),
        scratch_types=[
            pltpu.VMEM((CHUNKS, CHUNK), jnp.int32),
            pltpu.VMEM((MROWS, EMBED_DIM), jnp.float32),
            pltpu.VMEM((MROWS, EMBED_DIM), jnp.float32),
            pltpu.SemaphoreType.DMA,
            pltpu.SemaphoreType.DMA,
        ],
        compiler_params=pltpu.CompilerParams(use_tc_tiling_on_sc=False),
    )
    def gather_kernel(idx_hbm, table_hbm, out_hbm, idx_v, rows0, rows1, s0, s1):
        wid = lax.axis_index("s") * NUM_CORES + lax.axis_index("c")
        base = wid * BPW
        pltpu.sync_copy(idx_hbm.at[wid], idx_v)

        def fire(m, buf, sem):
            return [
                pltpu.async_copy(
                    table_hbm.at[idx_v.at[m * K + k]],
                    buf.at[pl.ds(k * CHUNK, CHUNK)],
                    sem,
                )
                for k in range(K)
            ]

        def wait_block(buf, sem):
            for k in range(K):
                pltpu.make_async_copy(
                    table_hbm.at[idx_v.at[k]],
                    buf.at[pl.ds(k * CHUNK, CHUNK)],
                    sem,
                ).wait()

        fire(0, rows0, s0)

        def body2(i, carry):
            m0 = 2 * i
            m1 = m0 + 1
            fire(m1, rows1, s1)
            wait_block(rows0, s0)
            pltpu.sync_copy(rows0, out_hbm.at[pl.ds(base + m0 * MROWS, MROWS)])

            @pl.when(i < NMACRO // 2 - 1)
            def _():
                fire(m0 + 2, rows0, s0)

            wait_block(rows1, s1)
            pltpu.sync_copy(rows1, out_hbm.at[pl.ds(base + m1 * MROWS, MROWS)])
            return carry

        lax.fori_loop(0, NMACRO // 2, body2, 0)

    return gather_kernel


_gather = _make_gather()


def kernel(x, table):
    idx = x.reshape(NW, CHUNKS, CHUNK).astype(jnp.int32)
    out = _gather(idx, table)
    return out.reshape(BATCH, HIST, EMBED_DIM)


# speedup vs baseline: 4.6835x; 1.0019x over previous
import functools

import jax
import jax.numpy as jnp
from jax import lax
from jax.experimental import pallas as pl
from jax.experimental.pallas import tpu as pltpu
from jax.experimental.pallas import tpu_sc as plsc

VOCAB = 100000
EMBED_DIM = 64
BATCH = 4096
HIST = 50

NUM_CORES = 2
NUM_SUBCORES = 16
NW = NUM_CORES * NUM_SUBCORES
TOTAL = BATCH * HIST
BPW = TOTAL // NW
CHUNK = 640
CHUNKS = BPW // CHUNK
K = 1
MROWS = K * CHUNK
NMACRO = CHUNKS // K


def _make_gather():
    mesh = plsc.VectorSubcoreMesh(core_axis_name="c", subcore_axis_name="s")

    @functools.partial(
        pl.kernel,
        mesh=mesh,
        out_type=jax.ShapeDtypeStruct((TOTAL, EMBED_DIM), jnp.float32),
        scratch_types=[
            pltpu.VMEM((CHUNKS, CHUNK), jnp.int32),
            pltpu.VMEM((MROWS, EMBED_DIM), jnp.float32),
            pltpu.VMEM((MROWS, EMBED_DIM), jnp.float32),
            pltpu.SemaphoreType.DMA,
            pltpu.SemaphoreType.DMA,
        ],
        compiler_params=pltpu.CompilerParams(use_tc_tiling_on_sc=False),
    )
    def gather_kernel(idx_hbm, table_hbm, out_hbm, idx_v, rows0, rows1, s0, s1):
        wid = lax.axis_index("s") * NUM_CORES + lax.axis_index("c")
        base = wid * BPW
        pltpu.sync_copy(idx_hbm.at[wid], idx_v)

        def fire(m, buf, sem):
            return [
                pltpu.async_copy(
                    table_hbm.at[idx_v.at[m * K + k]],
                    buf.at[pl.ds(k * CHUNK, CHUNK)],
                    sem,
                )
                for k in range(K)
            ]

        def wait_block(buf, sem):
            for k in range(K):
                pltpu.make_async_copy(
                    table_hbm.at[idx_v.at[k]],
                    buf.at[pl.ds(k * CHUNK, CHUNK)],
                    sem,
                ).wait()

        fire(0, rows0, s0)

        def body2(i, carry):
            m0 = 2 * i
            m1 = m0 + 1
            fire(m1, rows1, s1)
            wait_block(rows0, s0)
            pltpu.sync_copy(rows0, out_hbm.at[pl.ds(base + m0 * MROWS, MROWS)])

            @pl.when(i < NMACRO // 2 - 1)
            def _():
                fire(m0 + 2, rows0, s0)

            wait_block(rows1, s1)
            pltpu.sync_copy(rows1, out_hbm.at[pl.ds(base + m1 * MROWS, MROWS)])
            return carry

        lax.fori_loop(0, NMACRO // 2, body2, 0)

    return gather_kernel


_gather = _make_gather()


def kernel(x, table):
    idx = x.reshape(NW, CHUNKS, CHUNK).astype(jnp.int32)
    out = _gather(idx, table)
    return out.reshape(BATCH, HIST, EMBED_DIM)
